# baseline (speedup 1.0000x reference)
import jax
import jax.numpy as jnp
from jax import lax
from jax.experimental import pallas as pl
from jax.experimental.pallas import tpu as pltpu
from jax.experimental.pallas import tpu_sc as plsc

VOCAB = 100000
D = 768
S = 2048
B = 4
N = S * B
LN_EPS = 1e-12

NC = 2
NS = 16
NW = NC * NS
L = 16

PW = N // NW
C = 64
NCHUNK = PW // C
DV = D // L


def _ln_body(ids_hbm, word_hbm, pos_hbm, scale_hbm, bias_hbm, out_hbm,
             idx_v, buf, pbuf, scale_v, bias_v, sem):
    wid = lax.axis_index("s") * NC + lax.axis_index("c")
    base = wid * PW

    pltpu.sync_copy(scale_hbm, scale_v)
    pltpu.sync_copy(bias_hbm, bias_v)

    lane = lax.iota(jnp.int32, L)

    def vsum(x):
        for s in (8, 4, 2, 1):
            idx = lax.bitwise_xor(lane, s)
            x = x + x.at[idx].get(mode="promise_in_bounds", unique_indices=True)
        return x

    def chunk_step(g, _):
        row0 = pl.multiple_of(base + g * C, C)
        pltpu.sync_copy(ids_hbm.at[pl.ds(row0, C)], idx_v)
        pltpu.sync_copy(pos_hbm.at[pl.ds(pl.multiple_of(row0 // B, C // B), C // B)], pbuf)
        pltpu.async_copy(word_hbm.at[idx_v], buf, sem).wait()

        def token_step(i, carry):
            pi = lax.shift_right_logical(i, 2)
            acc_s = jnp.zeros((L,), jnp.float32)
            acc_q = jnp.zeros((L,), jnp.float32)
            for j in range(DV):
                sl = pl.ds(j * L, L)
                x = buf[i, sl] + pbuf[pi, sl]
                buf[i, sl] = x
                acc_s = acc_s + x
                acc_q = acc_q + x * x
            mean = vsum(acc_s) * (1.0 / D)
            var = vsum(acc_q) * (1.0 / D) - mean * mean
            v = var + LN_EPS
            yb = plsc.bitcast(v, jnp.int32)
            yb = 0x5F3759DF - jnp.right_shift(yb, 1)
            y = plsc.bitcast(yb, jnp.float32)
            h = 0.5 * v
            for _ in range(3):
                y = y * (1.5 - h * y * y)
            for j in range(DV):
                sl = pl.ds(j * L, L)
                x = buf[i, sl]
                buf[i, sl] = (x - mean) * y * scale_v[sl] + bias_v[sl]
            return carry

        lax.fori_loop(0, C, token_step, 0)
        pltpu.sync_copy(buf, out_hbm.at[pl.ds(row0, C)])
        return _

    lax.fori_loop(0, NCHUNK, chunk_step, 0)


@jax.jit
def kernel(input_ids, word_table, pos_table, ln_scale, ln_bias):
    ids = input_ids.reshape(N).astype(jnp.int32)
    mesh = plsc.VectorSubcoreMesh(core_axis_name="c", subcore_axis_name="s")
    run = pl.kernel(
        _ln_body,
        out_type=jax.ShapeDtypeStruct((N, D), jnp.float32),
        mesh=mesh,
        compiler_params=pltpu.CompilerParams(needs_layout_passes=False),
        scratch_types=[
            pltpu.VMEM((C,), jnp.int32),
            pltpu.VMEM((C, D), jnp.float32),
            pltpu.VMEM((C // B, D), jnp.float32),
            pltpu.VMEM((D,), jnp.float32),
            pltpu.VMEM((D,), jnp.float32),
            pltpu.SemaphoreType.DMA,
        ],
    )
    out = run(ids, word_table, pos_table, ln_scale, ln_bias)
    return out.reshape(S, B, D)
